# baseline (speedup 1.0000x reference)
import functools

import jax
import jax.numpy as jnp
from jax import lax
from jax.experimental import pallas as pl
from jax.experimental.pallas import tpu as pltpu
from jax.experimental.pallas import tpu_sc as plsc

NC = 2
NS = 16
NW = NC * NS
CW = 128
K = 2
DP = 128


@functools.lru_cache(maxsize=None)
def _build(n_total, d):
    per_w = n_total // NW
    ch = per_w // CW
    ng = ch // K
    nh = ng // 2

    mesh = plsc.VectorSubcoreMesh(core_axis_name="c", subcore_axis_name="s")

    @functools.partial(
        pl.kernel,
        out_type=jax.ShapeDtypeStruct((n_total, d), jnp.float32),
        mesh=mesh,
        scratch_types=[
            pltpu.VMEM((ch, CW), jnp.int32),
            pltpu.VMEM((2, K, CW, DP), jnp.float32),
            pltpu.SemaphoreType.DMA,
            pltpu.SemaphoreType.DMA,
        ],
        compiler_params=pltpu.CompilerParams(
            use_tc_tiling_on_sc=False, needs_layout_passes=False),
    )
    def k(ids_hbm, table_hbm, out_hbm, idx_v, bufs, gsem, wsem):
        wid = lax.axis_index("s") * NC + lax.axis_index("c")
        base = wid * ch
        pltpu.sync_copy(ids_hbm.at[wid], idx_v)

        def fire_gathers(g, s):
            for i in range(K):
                pltpu.async_copy(
                    table_hbm.at[idx_v.at[g * K + i]], bufs.at[s, i], gsem)

        def fire_writes(g, s):
            for i in range(K):
                pltpu.async_copy(
                    bufs.at[s, i, :, pl.ds(0, d)],
                    out_hbm.at[pl.ds((base + g * K + i) * CW, CW)], wsem)

        def drain_g(count):
            for _ in range(count):
                pltpu.make_async_copy(
                    table_hbm.at[pl.ds(0, CW)], bufs.at[0, 0], gsem).wait()

        def drain_w(count):
            for _ in range(count):
                pltpu.make_async_copy(
                    out_hbm.at[pl.ds(0, CW)],
                    bufs.at[0, 0, :, pl.ds(0, d)], wsem).wait()

        fire_gathers(0, 0)

        def body(h, carry):
            g0 = 2 * h
            g1 = g0 + 1

            @pl.when(h > 0)
            def _():
                drain_w(K)

            fire_gathers(g1, 1)
            drain_g(K)
            fire_writes(g0, 0)
            drain_g(K)
            fire_writes(g1, 1)
            drain_w(K)

            @pl.when(h + 1 < nh)
            def _():
                fire_gathers(g0 + 2, 0)

            return carry

        lax.fori_loop(0, nh, body, 0)
        drain_w(K)

    return k


def kernel(input_ids, table):
    b, s = input_ids.shape
    v, d = table.shape
    n = b * s
    ids = input_ids.astype(jnp.int32).reshape(NW, n // NW // CW, CW)
    table_p = jnp.pad(table, ((0, 0), (0, DP - d)))
    out = _build(n, d)(ids, table_p)
    return out.reshape(b, s, d)


# speedup vs baseline: 1.0333x; 1.0333x over previous
import functools

import jax
import jax.numpy as jnp
from jax import lax
from jax.experimental import pallas as pl
from jax.experimental.pallas import tpu as pltpu
from jax.experimental.pallas import tpu_sc as plsc

NC = 2
NS = 16
NW = NC * NS
CA = 128


@functools.lru_cache(maxsize=None)
def _build(b, s, d):
    bw = b // NW
    cb = s - CA
    nh = bw // 2

    mesh = plsc.VectorSubcoreMesh(core_axis_name="c", subcore_axis_name="s")

    @functools.partial(
        pl.kernel,
        out_type=jax.ShapeDtypeStruct((b, s, d), jnp.float32),
        mesh=mesh,
        scratch_types=[
            pltpu.VMEM((bw, s), jnp.int32),
            pltpu.VMEM((2, 2, CA, d), jnp.float32),
            pltpu.SemaphoreType.DMA,
            pltpu.SemaphoreType.DMA,
        ],
        compiler_params=pltpu.CompilerParams(use_tc_tiling_on_sc=False),
    )
    def k(ids_hbm, table_hbm, out_hbm, idx_v, bufs, gsem, wsem):
        wid = lax.axis_index("s") * NC + lax.axis_index("c")
        b0 = wid * bw
        pltpu.sync_copy(ids_hbm.at[pl.ds(b0, bw)], idx_v)

        def fire_gathers(g, st):
            pltpu.async_copy(
                table_hbm.at[idx_v.at[g, pl.ds(0, CA)]], bufs.at[st, 0], gsem)
            pltpu.async_copy(
                table_hbm.at[idx_v.at[g, pl.ds(CA, cb)]],
                bufs.at[st, 1, pl.ds(0, cb)], gsem)

        def fire_writes(g, st):
            pltpu.async_copy(
                bufs.at[st, 0], out_hbm.at[b0 + g, pl.ds(0, CA)], wsem)
            pltpu.async_copy(
                bufs.at[st, 1, pl.ds(0, cb)],
                out_hbm.at[b0 + g, pl.ds(CA, cb)], wsem)

        def drain(sem):
            pltpu.make_async_copy(
                table_hbm.at[pl.ds(0, CA)], bufs.at[0, 0], sem).wait()
            pltpu.make_async_copy(
                table_hbm.at[pl.ds(0, cb)],
                bufs.at[0, 1, pl.ds(0, cb)], sem).wait()

        fire_gathers(0, 0)

        def body(h, carry):
            g0 = 2 * h
            g1 = g0 + 1

            @pl.when(h > 0)
            def _():
                drain(wsem)

            fire_gathers(g1, 1)
            drain(gsem)
            fire_writes(g0, 0)
            drain(gsem)
            fire_writes(g1, 1)
            drain(wsem)

            @pl.when(h + 1 < nh)
            def _():
                fire_gathers(g0 + 2, 0)

            return carry

        lax.fori_loop(0, nh, body, 0)
        drain(wsem)

    return k


def kernel(input_ids, table):
    b, s = input_ids.shape
    v, d = table.shape
    return _build(b, s, d)(input_ids.astype(jnp.int32), table)
